# Initial kernel scaffold; baseline (speedup 1.0000x reference)
import jax
import jax.numpy as jnp
from jax.experimental import pallas as pl


def kernel(x0, x1, x2, W_agg0, b_agg0, W_self0, W_agg1, b_agg1, W_self1):
    raise NotImplementedError("write your pallas kernel here")


# speedup vs baseline: 1.8795x; 1.8795x over previous
import functools

import jax
import jax.numpy as jnp
from jax import lax
from jax.experimental import pallas as pl

F = 10


def _leaky(x):
    return jnp.where(x >= 0, x, 0.01 * x)


def _body(x0_r, x1_r, x1g_r, x2g_r, wa0_r, ba0_r, ws0_r, wa1_r, ba1_r, ws1_r,
          out_r, *, S):
    ws0 = ws0_r[...]
    wa0 = wa0_r[...]
    ba0 = ba0_r[...]

    m2 = x2g_r[:, 0, :]
    for k in range(1, F):
        m2 = m2 + x2g_r[:, k, :]
    m2 = m2 * (1.0 / F)

    m1 = x1g_r[:, 0, :]
    for k in range(1, F):
        m1 = m1 + x1g_r[:, k, :]
    m1 = m1 * (1.0 / F)

    h1 = (jnp.dot(x1_r[...], ws0, preferred_element_type=jnp.float32)
          + jnp.dot(m2, wa0, preferred_element_type=jnp.float32) + ba0)
    h1 = _leaky(h1)

    rows = lax.broadcasted_iota(jnp.int32, (S, S * F), 0)
    cols = lax.broadcasted_iota(jnp.int32, (S, S * F), 1)
    pool = jnp.where(cols // F == rows, 1.0 / F, 0.0)
    mh1 = jnp.dot(pool, h1, preferred_element_type=jnp.float32)

    h0 = (jnp.dot(x0_r[...], ws0, preferred_element_type=jnp.float32)
          + jnp.dot(m1, wa0, preferred_element_type=jnp.float32) + ba0)
    h0 = _leaky(h0)

    out_r[...] = (jnp.dot(h0, ws1_r[...], preferred_element_type=jnp.float32)
                  + jnp.dot(mh1, wa1_r[...], preferred_element_type=jnp.float32)
                  + ba1_r[...])


def kernel(x0, x1, x2, W_agg0, b_agg0, W_self0, W_agg1, b_agg1, W_self1):
    B, D_in = x0.shape
    D_h = W_agg0.shape[1]
    S = 40
    nb = B // S

    x1g = x1.reshape(B, F, D_in)
    x2g = x2.reshape(B * F, F, D_in)
    ba0 = b_agg0.reshape(1, D_h)
    ba1 = b_agg1.reshape(1, D_h)

    rep = lambda i: (0, 0)
    out = pl.pallas_call(
        functools.partial(_body, S=S),
        grid=(nb,),
        in_specs=[
            pl.BlockSpec((S, D_in), lambda i: (i, 0)),
            pl.BlockSpec((S * F, D_in), lambda i: (i, 0)),
            pl.BlockSpec((S, F, D_in), lambda i: (i, 0, 0)),
            pl.BlockSpec((S * F, F, D_in), lambda i: (i, 0, 0)),
            pl.BlockSpec((D_in, D_h), rep),
            pl.BlockSpec((1, D_h), rep),
            pl.BlockSpec((D_in, D_h), rep),
            pl.BlockSpec((D_h, D_h), rep),
            pl.BlockSpec((1, D_h), rep),
            pl.BlockSpec((D_h, D_h), rep),
        ],
        out_specs=pl.BlockSpec((S, D_h), lambda i: (i, 0)),
        out_shape=jax.ShapeDtypeStruct((B, D_h), jnp.float32),
    )(x0, x1, x1g, x2g, W_agg0, ba0, W_self0, W_agg1, ba1, W_self1)
    return out


# speedup vs baseline: 2.5800x; 1.3727x over previous
import functools

import jax
import jax.numpy as jnp
from jax.experimental import pallas as pl

F = 10


def _leaky(x):
    return jnp.where(x >= 0, x, 0.01 * x)


def _body(x0_r, x1g_r, x2g_r, wa0_r, ba0_r, ws0_r, wa1_r, ba1_r, ws1_r,
          out_r):
    ws0 = ws0_r[...]
    wa0 = wa0_r[...]
    ba0 = ba0_r[...]

    acc_h1 = None
    acc_m1 = None
    for k in range(F):
        m2k = x2g_r[:, F * k, :]
        for j in range(1, F):
            m2k = m2k + x2g_r[:, F * k + j, :]
        m2k = m2k * (1.0 / F)
        x1k = x1g_r[:, k, :]
        h1k = _leaky(jnp.dot(x1k, ws0, preferred_element_type=jnp.float32)
                     + jnp.dot(m2k, wa0, preferred_element_type=jnp.float32)
                     + ba0)
        acc_h1 = h1k if acc_h1 is None else acc_h1 + h1k
        acc_m1 = x1k if acc_m1 is None else acc_m1 + x1k
    mh1 = acc_h1 * (1.0 / F)
    m1 = acc_m1 * (1.0 / F)

    h0 = _leaky(jnp.dot(x0_r[...], ws0, preferred_element_type=jnp.float32)
                + jnp.dot(m1, wa0, preferred_element_type=jnp.float32) + ba0)

    out_r[...] = (jnp.dot(h0, ws1_r[...], preferred_element_type=jnp.float32)
                  + jnp.dot(mh1, wa1_r[...], preferred_element_type=jnp.float32)
                  + ba1_r[...])


def kernel(x0, x1, x2, W_agg0, b_agg0, W_self0, W_agg1, b_agg1, W_self1):
    B, D_in = x0.shape
    D_h = W_agg0.shape[1]
    S = 200
    nb = B // S

    x1g = x1.reshape(B, F, D_in)
    x2g = x2.reshape(B, F * F, D_in)
    ba0 = b_agg0.reshape(1, D_h)
    ba1 = b_agg1.reshape(1, D_h)

    rep = lambda i: (0, 0)
    out = pl.pallas_call(
        _body,
        grid=(nb,),
        in_specs=[
            pl.BlockSpec((S, D_in), lambda i: (i, 0)),
            pl.BlockSpec((S, F, D_in), lambda i: (i, 0, 0)),
            pl.BlockSpec((S, F * F, D_in), lambda i: (i, 0, 0)),
            pl.BlockSpec((D_in, D_h), rep),
            pl.BlockSpec((1, D_h), rep),
            pl.BlockSpec((D_in, D_h), rep),
            pl.BlockSpec((D_h, D_h), rep),
            pl.BlockSpec((1, D_h), rep),
            pl.BlockSpec((D_h, D_h), rep),
        ],
        out_specs=pl.BlockSpec((S, D_h), lambda i: (i, 0)),
        out_shape=jax.ShapeDtypeStruct((B, D_h), jnp.float32),
    )(x0, x1g, x2g, W_agg0, ba0, W_self0, W_agg1, ba1, W_self1)
    return out


# speedup vs baseline: 3.0013x; 1.1633x over previous
import functools

import jax
import jax.numpy as jnp
from jax.experimental import pallas as pl

F = 10


def _leaky(x):
    return jnp.where(x >= 0, x, 0.01 * x)


def _body(x0_r, x1g_r, x2g_r, wa0_r, ba0_r, ws0_r, wa1_r, ba1_r, ws1_r,
          out_r, *, d_in):
    ws0 = ws0_r[...]
    wa0 = wa0_r[...]
    ba0 = ba0_r[...]

    acc_h1 = None
    acc_m1 = None
    for k in range(F):
        m2k = x2g_r[:, F * k * d_in:(F * k + 1) * d_in]
        for j in range(1, F):
            c = (F * k + j) * d_in
            m2k = m2k + x2g_r[:, c:c + d_in]
        m2k = m2k * (1.0 / F)
        x1k = x1g_r[:, k * d_in:(k + 1) * d_in]
        h1k = _leaky(jnp.dot(x1k, ws0, preferred_element_type=jnp.float32)
                     + jnp.dot(m2k, wa0, preferred_element_type=jnp.float32)
                     + ba0)
        acc_h1 = h1k if acc_h1 is None else acc_h1 + h1k
        acc_m1 = x1k if acc_m1 is None else acc_m1 + x1k
    mh1 = acc_h1 * (1.0 / F)
    m1 = acc_m1 * (1.0 / F)

    h0 = _leaky(jnp.dot(x0_r[...], ws0, preferred_element_type=jnp.float32)
                + jnp.dot(m1, wa0, preferred_element_type=jnp.float32) + ba0)

    out_r[...] = (jnp.dot(h0, ws1_r[...], preferred_element_type=jnp.float32)
                  + jnp.dot(mh1, wa1_r[...], preferred_element_type=jnp.float32)
                  + ba1_r[...])


def kernel(x0, x1, x2, W_agg0, b_agg0, W_self0, W_agg1, b_agg1, W_self1):
    B, D_in = x0.shape
    D_h = W_agg0.shape[1]
    S = 200
    nb = B // S

    x1g = x1.reshape(B, F * D_in)
    x2g = x2.reshape(B, F * F * D_in)
    ba0 = b_agg0.reshape(1, D_h)
    ba1 = b_agg1.reshape(1, D_h)

    rep = lambda i: (0, 0)
    out = pl.pallas_call(
        functools.partial(_body, d_in=D_in),
        grid=(nb,),
        in_specs=[
            pl.BlockSpec((S, D_in), lambda i: (i, 0)),
            pl.BlockSpec((S, F * D_in), lambda i: (i, 0)),
            pl.BlockSpec((S, F * F * D_in), lambda i: (i, 0)),
            pl.BlockSpec((D_in, D_h), rep),
            pl.BlockSpec((1, D_h), rep),
            pl.BlockSpec((D_in, D_h), rep),
            pl.BlockSpec((D_h, D_h), rep),
            pl.BlockSpec((1, D_h), rep),
            pl.BlockSpec((D_h, D_h), rep),
        ],
        out_specs=pl.BlockSpec((S, D_h), lambda i: (i, 0)),
        out_shape=jax.ShapeDtypeStruct((B, D_h), jnp.float32),
    )(x0, x1g, x2g, W_agg0, ba0, W_self0, W_agg1, ba1, W_self1)
    return out


# speedup vs baseline: 3.0044x; 1.0010x over previous
import functools

import jax
import jax.numpy as jnp
from jax.experimental import pallas as pl

F = 10


def _leaky(x):
    return jnp.where(x >= 0, x, 0.01 * x)


def _body(x0_r, x1g_r, x2a_r, x2b_r, x2c_r, x2d_r, wa0_r, ba0_r, ws0_r,
          wa1_r, ba1_r, ws1_r, out_r, *, d_in):
    ws0 = ws0_r[...]
    wa0 = wa0_r[...]
    ba0 = ba0_r[...]
    parts = (x2a_r, x2b_r, x2c_r, x2d_r)
    npart = F * F // len(parts)

    def nslice(n):
        r = parts[n // npart]
        c = (n % npart) * d_in
        return r[:, c:c + d_in]

    acc_h1 = None
    acc_m1 = None
    for k in range(F):
        m2k = nslice(F * k)
        for j in range(1, F):
            m2k = m2k + nslice(F * k + j)
        m2k = m2k * (1.0 / F)
        x1k = x1g_r[:, k * d_in:(k + 1) * d_in]
        h1k = _leaky(jnp.dot(x1k, ws0, preferred_element_type=jnp.float32)
                     + jnp.dot(m2k, wa0, preferred_element_type=jnp.float32)
                     + ba0)
        acc_h1 = h1k if acc_h1 is None else acc_h1 + h1k
        acc_m1 = x1k if acc_m1 is None else acc_m1 + x1k
    mh1 = acc_h1 * (1.0 / F)
    m1 = acc_m1 * (1.0 / F)

    h0 = _leaky(jnp.dot(x0_r[...], ws0, preferred_element_type=jnp.float32)
                + jnp.dot(m1, wa0, preferred_element_type=jnp.float32) + ba0)

    out_r[...] = (jnp.dot(h0, ws1_r[...], preferred_element_type=jnp.float32)
                  + jnp.dot(mh1, wa1_r[...], preferred_element_type=jnp.float32)
                  + ba1_r[...])


def kernel(x0, x1, x2, W_agg0, b_agg0, W_self0, W_agg1, b_agg1, W_self1):
    B, D_in = x0.shape
    D_h = W_agg0.shape[1]
    S = 200
    nb = B // S

    x1g = x1.reshape(B, F * D_in)
    x2g = x2.reshape(B, F * F * D_in)
    ba0 = b_agg0.reshape(1, D_h)
    ba1 = b_agg1.reshape(1, D_h)

    wq = F * F * D_in // 4

    rep = lambda i: (0, 0)
    out = pl.pallas_call(
        functools.partial(_body, d_in=D_in),
        grid=(nb,),
        in_specs=[
            pl.BlockSpec((S, D_in), lambda i: (i, 0)),
            pl.BlockSpec((S, F * D_in), lambda i: (i, 0)),
            pl.BlockSpec((S, wq), lambda i: (i, 0)),
            pl.BlockSpec((S, wq), lambda i: (i, 1)),
            pl.BlockSpec((S, wq), lambda i: (i, 2)),
            pl.BlockSpec((S, wq), lambda i: (i, 3)),
            pl.BlockSpec((D_in, D_h), rep),
            pl.BlockSpec((1, D_h), rep),
            pl.BlockSpec((D_in, D_h), rep),
            pl.BlockSpec((D_h, D_h), rep),
            pl.BlockSpec((1, D_h), rep),
            pl.BlockSpec((D_h, D_h), rep),
        ],
        out_specs=pl.BlockSpec((S, D_h), lambda i: (i, 0)),
        out_shape=jax.ShapeDtypeStruct((B, D_h), jnp.float32),
    )(x0, x1g, x2g, x2g, x2g, x2g, W_agg0, ba0, W_self0, W_agg1, ba1,
      W_self1)
    return out


# speedup vs baseline: 3.7293x; 1.2413x over previous
import functools

import jax
import jax.numpy as jnp
from jax import lax
from jax.experimental import pallas as pl
from jax.experimental.pallas import tpu as pltpu
from jax.experimental.pallas import tpu_sc as plsc

F = 10
_NODES_PER_WORKER = 1568
_CHUNK = 32


def _leaky(x):
    return jnp.where(x >= 0, x, 0.01 * x)


def _sc_mean_body(x2_hbm, m2_hbm, buf0, buf1, ob0, ob1, sem0, sem1, osem0,
                  osem1):
    nc = plsc.get_sparse_core_info().num_cores
    wid = lax.axis_index("s") * nc + lax.axis_index("c")
    nstart = jnp.minimum(wid * _NODES_PER_WORKER,
                         m2_hbm.shape[0] - _NODES_PER_WORKER)
    nchunks = _NODES_PER_WORKER // _CHUNK
    bufs = (buf0, buf1)
    obufs = (ob0, ob1)
    sems = (sem0, sem1)
    osems = (osem0, osem1)

    def in_copy(g, slot):
        rbase = (nstart + g * _CHUNK) * F
        return pltpu.make_async_copy(
            x2_hbm.at[pl.ds(rbase, _CHUNK * F)], bufs[slot], sems[slot])

    def out_copy(g, slot):
        nbase = nstart + g * _CHUNK
        return pltpu.make_async_copy(
            obufs[slot], m2_hbm.at[pl.ds(nbase, _CHUNK)], osems[slot])

    in_copy(0, 0).start()

    def chunk(g, _):
        slot = lax.rem(g, 2)

        @pl.when(g + 1 < nchunks)
        def _():
            def start_next(s):
                in_copy(g + 1, s).start()
            lax.cond(slot == 0, lambda: start_next(1), lambda: start_next(0))

        def work(b, ob):
            def node(n, _):
                for col in range(8):
                    c = pl.ds(col * 16, 16)
                    acc = b[n * F, c]
                    for j in range(1, F):
                        acc = acc + b[n * F + j, c]
                    ob[n, c] = acc * (1.0 / F)
                return 0
            lax.fori_loop(0, _CHUNK, node, 0)

        def do_slot(s):
            in_copy(g, s).wait()

            @pl.when(g >= 2)
            def _():
                out_copy(g - 2, s).wait()

            work(bufs[s], obufs[s])
            out_copy(g, s).start()

        lax.cond(slot == 0, lambda: do_slot(0), lambda: do_slot(1))
        return 0

    lax.fori_loop(0, nchunks, chunk, 0)
    out_copy(nchunks - 2, (nchunks - 2) % 2).wait()
    out_copy(nchunks - 1, (nchunks - 1) % 2).wait()


def _sc_mean(x2):
    n_nodes = x2.shape[0] // F
    mesh = plsc.VectorSubcoreMesh(core_axis_name="c", subcore_axis_name="s")
    fn = pl.kernel(
        _sc_mean_body,
        mesh=mesh,
        out_type=jax.ShapeDtypeStruct((n_nodes, x2.shape[1]), jnp.float32),
        scratch_types=[
            pltpu.VMEM((_CHUNK * F, 128), jnp.float32),
            pltpu.VMEM((_CHUNK * F, 128), jnp.float32),
            pltpu.VMEM((_CHUNK, 128), jnp.float32),
            pltpu.VMEM((_CHUNK, 128), jnp.float32),
            pltpu.SemaphoreType.DMA,
            pltpu.SemaphoreType.DMA,
            pltpu.SemaphoreType.DMA,
            pltpu.SemaphoreType.DMA,
        ],
    )
    return fn(x2)


def _tc_body(x0_r, x1g_r, m2g_r, wa0_r, ba0_r, ws0_r, wa1_r, ba1_r, ws1_r,
             out_r, *, d_in):
    ws0 = ws0_r[...]
    wa0 = wa0_r[...]
    ba0 = ba0_r[...]

    acc_h1 = None
    acc_m1 = None
    for k in range(F):
        m2k = m2g_r[:, k * d_in:(k + 1) * d_in]
        x1k = x1g_r[:, k * d_in:(k + 1) * d_in]
        h1k = _leaky(jnp.dot(x1k, ws0, preferred_element_type=jnp.float32)
                     + jnp.dot(m2k, wa0, preferred_element_type=jnp.float32)
                     + ba0)
        acc_h1 = h1k if acc_h1 is None else acc_h1 + h1k
        acc_m1 = x1k if acc_m1 is None else acc_m1 + x1k
    mh1 = acc_h1 * (1.0 / F)
    m1 = acc_m1 * (1.0 / F)

    h0 = _leaky(jnp.dot(x0_r[...], ws0, preferred_element_type=jnp.float32)
                + jnp.dot(m1, wa0, preferred_element_type=jnp.float32) + ba0)

    out_r[...] = (jnp.dot(h0, ws1_r[...], preferred_element_type=jnp.float32)
                  + jnp.dot(mh1, wa1_r[...], preferred_element_type=jnp.float32)
                  + ba1_r[...])


def kernel(x0, x1, x2, W_agg0, b_agg0, W_self0, W_agg1, b_agg1, W_self1):
    B, D_in = x0.shape
    D_h = W_agg0.shape[1]
    S = 200
    nb = B // S

    m2 = _sc_mean(x2)

    x1g = x1.reshape(B, F * D_in)
    m2g = m2.reshape(B, F * D_in)
    ba0 = b_agg0.reshape(1, D_h)
    ba1 = b_agg1.reshape(1, D_h)

    rep = lambda i: (0, 0)
    out = pl.pallas_call(
        functools.partial(_tc_body, d_in=D_in),
        grid=(nb,),
        in_specs=[
            pl.BlockSpec((S, D_in), lambda i: (i, 0)),
            pl.BlockSpec((S, F * D_in), lambda i: (i, 0)),
            pl.BlockSpec((S, F * D_in), lambda i: (i, 0)),
            pl.BlockSpec((D_in, D_h), rep),
            pl.BlockSpec((1, D_h), rep),
            pl.BlockSpec((D_in, D_h), rep),
            pl.BlockSpec((D_h, D_h), rep),
            pl.BlockSpec((1, D_h), rep),
            pl.BlockSpec((D_h, D_h), rep),
        ],
        out_specs=pl.BlockSpec((S, D_h), lambda i: (i, 0)),
        out_shape=jax.ShapeDtypeStruct((B, D_h), jnp.float32),
    )(x0, x1g, m2g, W_agg0, ba0, W_self0, W_agg1, ba1, W_self1)
    return out
